# baseline (speedup 1.0000x reference)
import functools

import jax
import jax.numpy as jnp
from jax import lax
from jax.experimental import pallas as pl
from jax.experimental.pallas import tpu as pltpu
from jax.experimental.pallas import tpu_sc as plsc

N = 10000
IN_FEAT = 128
OUT_FEAT = 16

BI = 200
R_SC = 768
R_TC0 = 800
NTEC = 16
BR = 8
NBATCH = 6
KSPLIT = 4864
KSTAGE = 5120
KTAIL = 9984


def _proj_kernel(feat_ref, w_ref, adjtail_ref, adjrows_ref,
                 h_ref, tail_ref, brows_ref):
    h = jnp.dot(feat_ref[...], w_ref[...], preferred_element_type=jnp.float32)
    h_ref[...] = h
    tail_ref[...] = jnp.dot(adjtail_ref[...], h[KTAIL:, :],
                            preferred_element_type=jnp.float32)
    brows_ref[...] = jnp.dot(adjrows_ref[...], h,
                             preferred_element_type=jnp.float32)


def _tc_spmm_kernel(adj_ref, h_ref, out_ref):
    out_ref[...] = jnp.dot(adj_ref[...], h_ref[...],
                           preferred_element_type=jnp.float32)


def _sc_spmm_body(adj_hbm, ht_hbm, out_hbm, htbuf, abuf, obuf):
    c = lax.axis_index("c")
    s = lax.axis_index("s")
    k0 = c * KSPLIT
    nchunk = 304 + 16 * c

    pltpu.sync_copy(ht_hbm.at[:, pl.ds(k0, KSTAGE)], htbuf)

    iot = lax.iota(jnp.int32, 16)
    zero = jnp.zeros((16,), jnp.float32)
    rots = [(iot + (1 << b)) % 16 for b in range(4)]

    def _perm(v, idx):
        return lax.gather(
            v, idx[:, None],
            lax.GatherDimensionNumbers(offset_dims=(),
                                       collapsed_slice_dims=(0,),
                                       start_index_map=(0,)),
            slice_sizes=(1,),
            mode=lax.GatherScatterMode.PROMISE_IN_BOUNDS)

    def _lanesum(v):
        for idx in rots:
            v = v + _perm(v, idx)
        return v

    def row_batch(t, carry):
        rbase = 8 * (s + NTEC * t)
        pltpu.sync_copy(adj_hbm.at[pl.ds(rbase, BR), pl.ds(k0, KSTAGE)], abuf)

        for rp in range(BR // 2):
            def kbody(i, accs):
                off = i * 16
                a0 = abuf[2 * rp, pl.ds(off, 16)]
                a1 = abuf[2 * rp + 1, pl.ds(off, 16)]
                new = []
                for j in range(16):
                    ht = htbuf[j, pl.ds(off, 16)]
                    new.append(accs[j] + a0 * ht)
                    new.append(accs[16 + j] + a1 * ht)
                return tuple(new[0::2] + new[1::2])

            accs = lax.fori_loop(0, nchunk, kbody, (zero,) * 32)

            for r2 in range(2):
                out_row = zero
                for j in range(16):
                    sj = _lanesum(accs[r2 * 16 + j])
                    out_row = jnp.where(iot == j, sj, out_row)
                obuf[rp * 2 + r2, :] = out_row

        pltpu.sync_copy(obuf, out_hbm.at[pl.ds(c * R_SC + rbase, BR)])
        return carry

    lax.fori_loop(0, NBATCH, row_batch, 0)


@functools.partial(
    pl.kernel,
    mesh=plsc.VectorSubcoreMesh(core_axis_name="c", subcore_axis_name="s"),
    out_type=jax.ShapeDtypeStruct((2 * R_SC, OUT_FEAT), jnp.float32),
    scratch_types=[
        pltpu.VMEM((OUT_FEAT, KSTAGE), jnp.float32),
        pltpu.VMEM((BR, KSTAGE), jnp.float32),
        pltpu.VMEM((BR, OUT_FEAT), jnp.float32),
    ],
)
def _sc_spmm(adj_hbm, ht_hbm, out_hbm, htbuf, abuf, obuf):
    _sc_spmm_body(adj_hbm, ht_hbm, out_hbm, htbuf, abuf, obuf)


@jax.jit
def kernel(feat, adj, W):
    h, tail, brows = pl.pallas_call(
        _proj_kernel,
        grid=(1,),
        in_specs=[
            pl.BlockSpec((N, IN_FEAT), lambda i: (0, 0)),
            pl.BlockSpec((IN_FEAT, OUT_FEAT), lambda i: (0, 0)),
            pl.BlockSpec((R_SC, N - KTAIL), lambda i: (0, 0)),
            pl.BlockSpec((R_TC0 - R_SC, N), lambda i: (R_SC // (R_TC0 - R_SC), 0)),
        ],
        out_specs=(
            pl.BlockSpec((N, OUT_FEAT), lambda i: (0, 0)),
            pl.BlockSpec((R_SC, OUT_FEAT), lambda i: (0, 0)),
            pl.BlockSpec((R_TC0 - R_SC, OUT_FEAT), lambda i: (0, 0)),
        ),
        out_shape=(
            jax.ShapeDtypeStruct((N, OUT_FEAT), jnp.float32),
            jax.ShapeDtypeStruct((R_SC, OUT_FEAT), jnp.float32),
            jax.ShapeDtypeStruct((R_TC0 - R_SC, OUT_FEAT), jnp.float32),
        ),
    )(feat, W, jax.lax.slice(adj, (0, KTAIL), (R_SC, N)), adj)

    ht = h.T

    out_sc2 = _sc_spmm(adj, ht)
    out_sc = out_sc2[:R_SC] + out_sc2[R_SC:] + tail

    out_tc = pl.pallas_call(
        _tc_spmm_kernel,
        grid=((N - R_TC0) // BI,),
        in_specs=[
            pl.BlockSpec((BI, N), lambda i: (i + R_TC0 // BI, 0)),
            pl.BlockSpec((N, OUT_FEAT), lambda i: (0, 0)),
        ],
        out_specs=pl.BlockSpec((BI, OUT_FEAT), lambda i: (i, 0)),
        out_shape=jax.ShapeDtypeStruct((N - R_TC0, OUT_FEAT), jnp.float32),
    )(adj, h)

    return (jnp.concatenate([out_sc, brows, out_tc], axis=0),
            feat.shape[0] - 1)


# speedup vs baseline: 1.0066x; 1.0066x over previous
import functools

import jax
import jax.numpy as jnp
from jax import lax
from jax.experimental import pallas as pl
from jax.experimental.pallas import tpu as pltpu
from jax.experimental.pallas import tpu_sc as plsc

N = 10000
IN_FEAT = 128
OUT_FEAT = 16

BI = 200
R_SC = 960
R_TC0 = 1000
NTEC = 16
BR = 8
NBATCH = 15
RGROUP = R_SC // 8
KW = 2432
NCHUNK = KW // 16
KTAIL = 4 * KW


def _proj_kernel(feat_ref, w_ref, adjtail_ref, adjrows_ref,
                 h_ref, ht_ref, tail_ref, brows_ref):
    h = jnp.dot(feat_ref[...], w_ref[...], preferred_element_type=jnp.float32)
    h_ref[...] = h
    ht_ref[...] = h.T
    tail_ref[...] = jnp.dot(adjtail_ref[...], h[KTAIL:, :],
                            preferred_element_type=jnp.float32)
    brows_ref[...] = jnp.dot(adjrows_ref[...], h,
                             preferred_element_type=jnp.float32)


def _tc_spmm_kernel(adj_ref, h_ref, out_ref):
    out_ref[...] = jnp.dot(adj_ref[...], h_ref[...],
                           preferred_element_type=jnp.float32)


def _sc_spmm_body(adj_hbm, ht_hbm, out_hbm, htbuf, abuf, obuf, sem0, sem1):
    c = lax.axis_index("c")
    s = lax.axis_index("s")
    kq = 2 * c + (s % 2)
    sr = s // 2
    k0 = kq * KW
    row_base = sr * RGROUP

    pltpu.sync_copy(ht_hbm.at[:, pl.ds(k0, KW)], htbuf)

    iot = lax.iota(jnp.int32, 16)
    zero = jnp.zeros((16,), jnp.float32)
    rots = [(iot + (1 << b)) % 16 for b in range(4)]

    def _perm(v, idx):
        return lax.gather(
            v, idx[:, None],
            lax.GatherDimensionNumbers(offset_dims=(),
                                       collapsed_slice_dims=(0,),
                                       start_index_map=(0,)),
            slice_sizes=(1,),
            mode=lax.GatherScatterMode.PROMISE_IN_BOUNDS)

    def _lanesum(v):
        for idx in rots:
            v = v + _perm(v, idx)
        return v

    def _issue(b, buf, sem):
        rb = row_base + b * BR
        return pltpu.make_async_copy(
            adj_hbm.at[pl.ds(rb, BR), pl.ds(k0, KW)], buf, sem)

    def _compute(b, buf):
        for rp in range(BR // 2):
            def kbody(i, accs):
                off = i * 16
                a0 = buf[2 * rp, pl.ds(off, 16)]
                a1 = buf[2 * rp + 1, pl.ds(off, 16)]
                new0, new1 = [], []
                for j in range(16):
                    ht = htbuf[j, pl.ds(off, 16)]
                    new0.append(accs[j] + a0 * ht)
                    new1.append(accs[16 + j] + a1 * ht)
                return tuple(new0 + new1)

            accs = lax.fori_loop(0, NCHUNK, kbody, (zero,) * 32)

            for r2 in range(2):
                out_row = zero
                for j in range(16):
                    sj = _lanesum(accs[r2 * 16 + j])
                    out_row = jnp.where(iot == j, sj, out_row)
                obuf[rp * 2 + r2, :] = out_row
        pltpu.sync_copy(
            obuf, out_hbm.at[pl.ds(kq * R_SC + row_base + b * BR, BR)])

    _issue(0, abuf.at[0], sem0).start()

    def pair(tt, carry):
        b0 = 2 * tt
        _issue(b0, abuf.at[0], sem0).wait()
        _issue(b0 + 1, abuf.at[1], sem1).start()
        _compute(b0, abuf.at[0])
        _issue(b0 + 1, abuf.at[1], sem1).wait()
        _issue(b0 + 2, abuf.at[0], sem0).start()
        _compute(b0 + 1, abuf.at[1])
        return carry

    lax.fori_loop(0, (NBATCH - 1) // 2, pair, 0)

    _issue(NBATCH - 1, abuf.at[0], sem0).wait()
    _compute(NBATCH - 1, abuf.at[0])


@functools.partial(
    pl.kernel,
    mesh=plsc.VectorSubcoreMesh(core_axis_name="c", subcore_axis_name="s"),
    out_type=jax.ShapeDtypeStruct((4 * R_SC, OUT_FEAT), jnp.float32),
    scratch_types=[
        pltpu.VMEM((OUT_FEAT, KW), jnp.float32),
        pltpu.VMEM((2, BR, KW), jnp.float32),
        pltpu.VMEM((BR, OUT_FEAT), jnp.float32),
        pltpu.SemaphoreType.DMA,
        pltpu.SemaphoreType.DMA,
    ],
)
def _sc_spmm(adj_hbm, ht_hbm, out_hbm, htbuf, abuf, obuf, sem0, sem1):
    _sc_spmm_body(adj_hbm, ht_hbm, out_hbm, htbuf, abuf, obuf, sem0, sem1)


@jax.jit
def kernel(feat, adj, W):
    h, ht, tail, brows = pl.pallas_call(
        _proj_kernel,
        grid=(1,),
        in_specs=[
            pl.BlockSpec((N, IN_FEAT), lambda i: (0, 0)),
            pl.BlockSpec((IN_FEAT, OUT_FEAT), lambda i: (0, 0)),
            pl.BlockSpec((R_SC, N - KTAIL), lambda i: (0, 0)),
            pl.BlockSpec((R_TC0 - R_SC, N), lambda i: (R_SC // (R_TC0 - R_SC), 0)),
        ],
        out_specs=(
            pl.BlockSpec((N, OUT_FEAT), lambda i: (0, 0)),
            pl.BlockSpec((OUT_FEAT, N), lambda i: (0, 0)),
            pl.BlockSpec((R_SC, OUT_FEAT), lambda i: (0, 0)),
            pl.BlockSpec((R_TC0 - R_SC, OUT_FEAT), lambda i: (0, 0)),
        ),
        out_shape=(
            jax.ShapeDtypeStruct((N, OUT_FEAT), jnp.float32),
            jax.ShapeDtypeStruct((OUT_FEAT, N), jnp.float32),
            jax.ShapeDtypeStruct((R_SC, OUT_FEAT), jnp.float32),
            jax.ShapeDtypeStruct((R_TC0 - R_SC, OUT_FEAT), jnp.float32),
        ),
    )(feat, W, jax.lax.slice(adj, (0, KTAIL), (R_SC, N)), adj)

    out_sc4 = _sc_spmm(adj, ht)
    out_sc = (out_sc4[:R_SC] + out_sc4[R_SC:2 * R_SC]
              + out_sc4[2 * R_SC:3 * R_SC] + out_sc4[3 * R_SC:] + tail)

    out_tc = pl.pallas_call(
        _tc_spmm_kernel,
        grid=((N - R_TC0) // BI,),
        in_specs=[
            pl.BlockSpec((BI, N), lambda i: (i + R_TC0 // BI, 0)),
            pl.BlockSpec((N, OUT_FEAT), lambda i: (0, 0)),
        ],
        out_specs=pl.BlockSpec((BI, OUT_FEAT), lambda i: (i, 0)),
        out_shape=jax.ShapeDtypeStruct((N - R_TC0, OUT_FEAT), jnp.float32),
    )(adj, h)

    return (jnp.concatenate([out_sc, brows, out_tc], axis=0),
            feat.shape[0] - 1)


# speedup vs baseline: 1.2756x; 1.2672x over previous
import jax
import jax.numpy as jnp
from jax.experimental import pallas as pl
from jax.experimental.pallas import tpu as pltpu

N = 10000
IN_FEAT = 128
OUT_FEAT = 16
BI = 200


def _fused_kernel(feat_ref, w_ref, adj_ref, out_ref, h_ref):
    @pl.when(pl.program_id(0) == 0)
    def _():
        h_ref[...] = jnp.dot(feat_ref[...], w_ref[...],
                             preferred_element_type=jnp.float32)

    out_ref[...] = jnp.dot(adj_ref[...], h_ref[...],
                           preferred_element_type=jnp.float32)


@jax.jit
def kernel(feat, adj, W):
    out = pl.pallas_call(
        _fused_kernel,
        grid=(N // BI,),
        in_specs=[
            pl.BlockSpec((N, IN_FEAT), lambda i: (0, 0)),
            pl.BlockSpec((IN_FEAT, OUT_FEAT), lambda i: (0, 0)),
            pl.BlockSpec((BI, N), lambda i: (i, 0)),
        ],
        out_specs=pl.BlockSpec((BI, OUT_FEAT), lambda i: (i, 0)),
        out_shape=jax.ShapeDtypeStruct((N, OUT_FEAT), jnp.float32),
        scratch_shapes=[pltpu.VMEM((N, OUT_FEAT), jnp.float32)],
    )(feat, W, adj)
    return (out, feat.shape[0] - 1)
